# Initial kernel scaffold; baseline (speedup 1.0000x reference)
import jax
import jax.numpy as jnp
from jax.experimental import pallas as pl


def kernel(inputs):
    raise NotImplementedError("write your pallas kernel here")


# speedup vs baseline: 2.4919x; 2.4919x over previous
import functools

import jax
import jax.numpy as jnp
from jax import lax
from jax.experimental import pallas as pl
from jax.experimental.pallas import tpu as pltpu
from jax.experimental.pallas import tpu_sc as plsc

_BATCH = 1024
_ND = 100
_NS = 20
_NCH = 1000
_ROW = _ND + _ND * _NS
_ROW_PAD = 2112
_OUT_W = _NCH * _NS
_NCORES = 2
_NSUB = 16
_NW = _NCORES * _NSUB
_RPW = _BATCH // _NW
_L = 16


def _body(in_hbm, out_hbm, row_v, dense_v, sem):
    cid = lax.axis_index("c")
    sid = lax.axis_index("s")
    wid = sid * _NCORES + cid

    lane = lax.iota(jnp.int32, _L)
    mask_tail = lane < (_NS - _L)
    zeros = jnp.zeros((_L,), jnp.float32)

    def _z(i, carry):
        dense_v[pl.ds(i * _L, _L)] = zeros
        return carry
    lax.fori_loop(0, _OUT_W // _L, _z, 0)

    def _bases(c):
        idxf = row_v[pl.ds(c * _L, _L)]
        return idxf.astype(jnp.int32) * _NS

    def _addr(bases, j):
        b = bases.at[jnp.full((_L,), j, jnp.int32)].get(
            mode="promise_in_bounds")
        return b + lane

    def _row(i, carry):
        r = wid * _RPW + i
        pltpu.sync_copy(in_hbm.at[r], row_v.at[pl.ds(0, _ROW_PAD)])
        for c in range(7):
            nvalid = _L if c < 6 else _ND - 6 * _L
            bases = _bases(c)
            for j in range(nvalid):
                d = c * _L + j
                a1 = _addr(bases, j)
                v1 = row_v[pl.ds(_ND + d * _NS, _L)]
                v2 = row_v[pl.ds(_ND + d * _NS + _L, _L)]
                plsc.addupdate_scatter(dense_v, [a1], v1)
                plsc.addupdate_scatter(dense_v, [a1 + _L], v2,
                                       mask=mask_tail)
        pltpu.sync_copy(dense_v, out_hbm.at[r])
        for c in range(7):
            nvalid = _L if c < 6 else _ND - 6 * _L
            bases = _bases(c)
            for j in range(nvalid):
                a1 = _addr(bases, j)
                plsc.store_scatter(dense_v, [a1], zeros)
                plsc.store_scatter(dense_v, [a1 + _L], zeros,
                                   mask=mask_tail)
        return carry

    lax.fori_loop(0, _RPW, _row, 0)


def kernel(inputs):
    x = jnp.pad(inputs, ((0, 0), (0, _ROW_PAD - _ROW)))
    mesh = plsc.VectorSubcoreMesh(
        core_axis_name="c", subcore_axis_name="s",
        num_cores=_NCORES, num_subcores=_NSUB)
    run = pl.kernel(
        _body,
        out_type=jax.ShapeDtypeStruct((_BATCH, _OUT_W), jnp.float32),
        mesh=mesh,
        compiler_params=pltpu.CompilerParams(
            use_tc_tiling_on_sc=False, needs_layout_passes=False),
        scratch_types=[
            pltpu.VMEM((_ROW_PAD,), jnp.float32),
            pltpu.VMEM((_OUT_W,), jnp.float32),
            pltpu.SemaphoreType.DMA,
        ],
    )
    out = run(x)
    return out.reshape(_BATCH, _NCH, _NS)[..., None]
